# baseline (speedup 1.0000x reference)
import functools
import math

import jax
import jax.numpy as jnp
from jax import lax
from jax.experimental import pallas as pl
from jax.experimental.pallas import tpu as pltpu
from jax.experimental.pallas import tpu_sc as plsc

NC = 2
NS = 16
NW = NC * NS
L = 16


@functools.partial(jax.jit, static_argnames=("B", "S", "D", "C"))
def _posemb_sc(idx3, word_table, pos_table, B, S, D, C):
    N = B * S
    n_per_w = N // NW
    n_chunks = n_per_w // C
    scale = float(math.sqrt(D))

    mesh = plsc.VectorSubcoreMesh(core_axis_name="c", subcore_axis_name="s")

    @functools.partial(
        pl.kernel,
        mesh=mesh,
        out_type=jax.ShapeDtypeStruct((N, D), jnp.float32),
        scratch_types=[
            pltpu.VMEM((n_chunks, C), jnp.int32),
            pltpu.VMEM((2, C, D), jnp.float32),
            pltpu.VMEM((2, C, D), jnp.float32),
            pltpu.SemaphoreType.DMA,
            pltpu.SemaphoreType.DMA,
            pltpu.SemaphoreType.DMA,
            pltpu.SemaphoreType.DMA,
        ],
    )
    def run(idx_hbm, word_hbm, pos_hbm, out_hbm, idx_v, wbuf, pbuf,
            wsem0, wsem1, psem0, psem1):
        cid = lax.axis_index("c")
        sid = lax.axis_index("s")
        wid = sid * NC + cid
        base = wid * n_per_w
        s_base = base % S

        pltpu.sync_copy(idx_hbm.at[wid], idx_v)

        wsems = (wsem0, wsem1)
        psems = (psem0, psem1)

        def start(g):
            slot = g % 2
            hw = pltpu.async_copy(word_hbm.at[idx_v.at[g]], wbuf.at[slot],
                                  wsems[slot])
            hp = pltpu.async_copy(pos_hbm.at[pl.ds(s_base + g * C, C)],
                                  pbuf.at[slot], psems[slot])
            return hw, hp

        pending = start(0)
        for g in range(n_chunks):
            slot = g % 2
            hw, hp = pending
            if g + 1 < n_chunks:
                nxt = start(g + 1)
            hw.wait()
            hp.wait()

            def row(i, _):
                def col(j, _):
                    w = wbuf[slot, i, pl.ds(j * L, L)]
                    p = pbuf[slot, i, pl.ds(j * L, L)]
                    wbuf[slot, i, pl.ds(j * L, L)] = w * scale + p
                    return 0
                return lax.fori_loop(0, D // L, col, 0)

            lax.fori_loop(0, C, row, 0)

            pltpu.sync_copy(wbuf.at[slot],
                            out_hbm.at[pl.ds(base + g * C, C)])
            if g + 1 < n_chunks:
                pending = nxt

    return run(idx3, word_table, pos_table)


def kernel(x, word_table, pos_table):
    B, S = x.shape
    V, D = word_table.shape
    N = B * S
    C = 16
    n_per_w = N // NW
    assert N % (NW * C) == 0
    assert S % n_per_w == 0
    idx3 = x.reshape(NW, n_per_w // C, C).astype(jnp.int32)
    out = _posemb_sc(idx3, word_table, pos_table, B, S, D, C)
    return out.reshape(B, S, D)


# speedup vs baseline: 2.0895x; 2.0895x over previous
import functools
import math

import jax
import jax.numpy as jnp
from jax import lax
from jax.experimental import pallas as pl
from jax.experimental.pallas import tpu as pltpu
from jax.experimental.pallas import tpu_sc as plsc

NC = 2
NS = 16
NW = NC * NS
L = 16


@functools.partial(jax.jit, static_argnames=("B", "S", "D", "C"))
def _posemb_sc(idx3, word_table, pos_table, B, S, D, C):
    N = B * S
    n_per_w = N // NW
    n_chunks = n_per_w // C
    scale = float(math.sqrt(D))

    mesh = plsc.VectorSubcoreMesh(core_axis_name="c", subcore_axis_name="s")

    @functools.partial(
        pl.kernel,
        mesh=mesh,
        out_type=jax.ShapeDtypeStruct((N, D), jnp.float32),
        scratch_types=[
            pltpu.VMEM((n_chunks, C), jnp.int32),
            pltpu.VMEM((2, C, D), jnp.float32),
            pltpu.VMEM((2, C, D), jnp.float32),
            pltpu.SemaphoreType.DMA,
            pltpu.SemaphoreType.DMA,
            pltpu.SemaphoreType.DMA,
            pltpu.SemaphoreType.DMA,
            pltpu.SemaphoreType.DMA,
            pltpu.SemaphoreType.DMA,
        ],
    )
    def run(idx_hbm, word_hbm, pos_hbm, out_hbm, idx_v, wbuf, pbuf,
            wsem0, wsem1, psem0, psem1, osem0, osem1):
        cid = lax.axis_index("c")
        sid = lax.axis_index("s")
        wid = sid * NC + cid
        base = wid * n_per_w
        s_base = base % S

        pltpu.sync_copy(idx_hbm.at[wid], idx_v)

        wsems = (wsem0, wsem1)
        psems = (psem0, psem1)
        osems = (osem0, osem1)

        def start(g):
            slot = g % 2
            hw = pltpu.async_copy(word_hbm.at[idx_v.at[g]], wbuf.at[slot],
                                  wsems[slot])
            hp = pltpu.async_copy(pos_hbm.at[pl.ds(s_base + g * C, C)],
                                  pbuf.at[slot], psems[slot])
            return hw, hp

        vregs = D // L
        owaits = [None, None]
        pending = start(0)
        for g in range(n_chunks):
            slot = g % 2
            if g + 1 < n_chunks:
                nslot = (g + 1) % 2
                if owaits[nslot] is not None:
                    owaits[nslot].wait()
                    owaits[nslot] = None
                nxt = start(g + 1)
            hw, hp = pending
            hw.wait()
            hp.wait()

            @plsc.parallel_loop(0, C * vregs, unroll=8)
            def _(k):
                i = k // vregs
                j = (k % vregs) * L
                w = wbuf[slot, i, pl.ds(j, L)]
                p = pbuf[slot, i, pl.ds(j, L)]
                wbuf[slot, i, pl.ds(j, L)] = w * scale + p

            owaits[slot] = pltpu.async_copy(
                wbuf.at[slot], out_hbm.at[pl.ds(base + g * C, C)],
                osems[slot])
            if g + 1 < n_chunks:
                pending = nxt
        for h in owaits:
            if h is not None:
                h.wait()

    return run(idx3, word_table, pos_table)


def kernel(x, word_table, pos_table):
    B, S = x.shape
    V, D = word_table.shape
    N = B * S
    C = 16
    n_per_w = N // NW
    assert N % (NW * C) == 0
    assert S % n_per_w == 0
    idx3 = x.reshape(NW, n_per_w // C, C).astype(jnp.int32)
    out = _posemb_sc(idx3, word_table, pos_table, B, S, D, C)
    return out.reshape(B, S, D)


# speedup vs baseline: 2.4015x; 1.1493x over previous
import functools
import math

import jax
import jax.numpy as jnp
from jax import lax
from jax.experimental import pallas as pl
from jax.experimental.pallas import tpu as pltpu
from jax.experimental.pallas import tpu_sc as plsc

NC = 2
NS = 16
NW = NC * NS
L = 16


@functools.partial(jax.jit, static_argnames=("B", "S", "D", "Cs"))
def _posemb_sc(idx4, word_table, pos_table, B, S, D, Cs):
    N = B * S
    s_per_w = S // NW
    n_chunks = s_per_w // Cs
    R = B * Cs
    scale = float(math.sqrt(D))
    vregs = D // L

    mesh = plsc.VectorSubcoreMesh(core_axis_name="c", subcore_axis_name="s")

    @functools.partial(
        pl.kernel,
        mesh=mesh,
        out_type=jax.ShapeDtypeStruct((N, D), jnp.float32),
        scratch_types=[
            pltpu.VMEM((n_chunks, R), jnp.int32),
            pltpu.VMEM((2, R, D), jnp.float32),
            pltpu.VMEM((2, Cs, D), jnp.float32),
            pltpu.SemaphoreType.DMA,
            pltpu.SemaphoreType.DMA,
            pltpu.SemaphoreType.DMA,
            pltpu.SemaphoreType.DMA,
            pltpu.SemaphoreType.DMA,
            pltpu.SemaphoreType.DMA,
        ],
    )
    def run(idx_hbm, word_hbm, pos_hbm, out_hbm, idx_v, wbuf, pbuf,
            wsem0, wsem1, psem0, psem1, osem0, osem1):
        cid = lax.axis_index("c")
        sid = lax.axis_index("s")
        wid = sid * NC + cid
        s_base = wid * s_per_w

        pltpu.sync_copy(idx_hbm.at[wid], idx_v)

        wsems = (wsem0, wsem1)
        psems = (psem0, psem1)
        osems = (osem0, osem1)

        def start(g):
            slot = g % 2
            hw = pltpu.async_copy(word_hbm.at[idx_v.at[g]], wbuf.at[slot],
                                  wsems[slot])
            hp = pltpu.async_copy(pos_hbm.at[pl.ds(s_base + g * Cs, Cs)],
                                  pbuf.at[slot], psems[slot])
            return hw, hp

        owaits = [None, None]
        pending = start(0)
        for g in range(n_chunks):
            slot = g % 2
            if g + 1 < n_chunks:
                nslot = (g + 1) % 2
                for h in owaits[nslot] or ():
                    h.wait()
                owaits[nslot] = None
                nxt = start(g + 1)
            hw, hp = pending
            hw.wait()
            hp.wait()

            @plsc.parallel_loop(0, R * vregs, unroll=8)
            def _(k):
                r = k // vregs
                j = (k % vregs) * L
                i = r % Cs
                w = wbuf[slot, r, pl.ds(j, L)]
                p = pbuf[slot, i, pl.ds(j, L)]
                wbuf[slot, r, pl.ds(j, L)] = w * scale + p

            os_ = []
            for b in range(B):
                os_.append(pltpu.async_copy(
                    wbuf.at[slot, pl.ds(b * Cs, Cs)],
                    out_hbm.at[pl.ds(b * S + s_base + g * Cs, Cs)],
                    osems[slot]))
            owaits[slot] = os_
            if g + 1 < n_chunks:
                pending = nxt
        for hs in owaits:
            for h in hs or ():
                h.wait()

    return run(idx4, word_table, pos_table)


def kernel(x, word_table, pos_table):
    B, S = x.shape
    V, D = word_table.shape
    N = B * S
    Cs = 8
    s_per_w = S // NW
    assert S % (NW * Cs) == 0 and D % L == 0
    idx4 = (x.reshape(B, NW, s_per_w // Cs, Cs)
              .transpose(1, 2, 0, 3)
              .reshape(NW, s_per_w // Cs, B * Cs)
              .astype(jnp.int32))
    out = _posemb_sc(idx4, word_table, pos_table, B, S, D, Cs)
    return out.reshape(B, S, D)
